# Initial kernel scaffold; baseline (speedup 1.0000x reference)
import jax
import jax.numpy as jnp
from jax.experimental import pallas as pl


def kernel(input, per_sample_weights, offsets, weight):
    raise NotImplementedError("write your pallas kernel here")


# speedup vs baseline: 1.9433x; 1.9433x over previous
import functools

import jax
import jax.numpy as jnp
from jax import lax
from jax.experimental import pallas as pl
from jax.experimental.pallas import tpu as pltpu
from jax.experimental.pallas import tpu_sc as plsc


def _full16(v):
    return jnp.full((16,), v, dtype=jnp.int32)


def _make_sc_kernel(B, N, M, V, D, NW, BPW):
    NC = 2
    NP = N + 8
    NH = N // 2
    NG = NP // 16

    mesh = plsc.VectorSubcoreMesh(core_axis_name="c", subcore_axis_name="s")

    @functools.partial(
        pl.kernel,
        out_type=jax.ShapeDtypeStruct((B, M, D), jnp.float32),
        mesh=mesh,
        compiler_params=pltpu.CompilerParams(
            needs_layout_passes=False, use_tc_tiling_on_sc=False),
        scratch_types=[
            pltpu.VMEM((BPW, 2, NH), jnp.int32),
            pltpu.VMEM((2, NP * 16), jnp.float32),
            pltpu.VMEM((BPW * 32,), jnp.int32),
            pltpu.VMEM((2, NP, D), jnp.float32),
            pltpu.VMEM(((NP + 1) * D,), jnp.float32),
            pltpu.VMEM((2, M, D), jnp.float32),
            pltpu.SemaphoreType.DMA,
            pltpu.SemaphoreType.DMA,
        ],
    )
    def sc_kernel(inp_hbm, pswx_hbm, bnd_hbm, w_hbm, out_hbm,
                  idx_v, pswx_v, bnd_v, rows_v, p_v, ost_v, gsem, osem):
        wid = lax.axis_index("s") * NC + lax.axis_index("c")
        base = wid * BPW
        zero16 = jnp.zeros((16,), jnp.float32)
        col0 = lax.iota(jnp.int32, 16)

        pltpu.sync_copy(inp_hbm.at[pl.ds(base, BPW)], idx_v)
        pltpu.sync_copy(bnd_hbm.at[pl.ds(base * 32, BPW * 32)], bnd_v)

        for bufc in range(2):
            for j in range(NP - N):
                for c in range(D // 16):
                    rows_v[bufc, N + j, pl.ds(c * 16, 16)] = zero16

        def issue_gather(i, buf):
            b = base + i
            pltpu.async_copy(pswx_hbm.at[pl.ds(b * NP * 16, NP * 16)],
                             pswx_v.at[buf], gsem)
            pltpu.async_copy(w_hbm.at[idx_v.at[i, 0]],
                             rows_v.at[buf, pl.ds(0, NH)], gsem)
            pltpu.async_copy(w_hbm.at[idx_v.at[i, 1]],
                             rows_v.at[buf, pl.ds(NH, NH)], gsem)

        def wait_gather(i, buf):
            b = base + i
            pltpu.make_async_copy(pswx_hbm.at[pl.ds(b * NP * 16, NP * 16)],
                                  pswx_v.at[buf], gsem).wait()
            pltpu.make_async_copy(w_hbm.at[idx_v.at[i, 0]],
                                  rows_v.at[buf, pl.ds(0, NH)], gsem).wait()
            pltpu.make_async_copy(w_hbm.at[idx_v.at[i, 1]],
                                  rows_v.at[buf, pl.ds(NH, NH)], gsem).wait()

        issue_gather(0, 0)

        def batch_body(i, carry):
            buf = lax.rem(i, 2)
            b = base + i
            wait_gather(i, buf)

            @pl.when(i + 1 < BPW)
            def _():
                issue_gather(i + 1, 1 - buf)

            @pl.when(i >= 2)
            def _():
                pltpu.make_async_copy(ost_v.at[buf], out_hbm.at[b], osem).wait()

            for c in range(D // 16):
                p_v[pl.ds(c * 16, 16)] = zero16

            def gbody(g, acc):
                acc = list(acc)
                nb = g * 16
                for j in range(0, 16, 2):
                    n0 = nb + j
                    n1 = nb + j + 1
                    pw0 = pswx_v[buf, pl.ds(n0 * 16, 16)]
                    pw1 = pswx_v[buf, pl.ds(n1 * 16, 16)]
                    r0 = [rows_v[buf, n0, pl.ds(c * 16, 16)]
                          for c in range(D // 16)]
                    r1 = [rows_v[buf, n1, pl.ds(c * 16, 16)]
                          for c in range(D // 16)]
                    for c in range(D // 16):
                        acc[c] = acc[c] + r0[c] * pw0
                        p_v[pl.ds((n0 + 1) * D + c * 16, 16)] = acc[c]
                    for c in range(D // 16):
                        acc[c] = acc[c] + r1[c] * pw1
                        p_v[pl.ds((n1 + 1) * D + c * 16, 16)] = acc[c]
                return tuple(acc)

            lax.fori_loop(0, NG, gbody, (zero16,) * (D // 16))

            bbase = i * 32
            brow = [plsc.load_gather(bnd_v, [_full16(bbase + m + 1)]) * D
                    for m in range(M)]
            pv = [[plsc.load_gather(p_v, [brow[m] + col0 + c * 16])
                   for c in range(D // 16)] for m in range(M)]
            for m in range(M):
                for c in range(D // 16):
                    if m == 0:
                        ost_v[buf, m, pl.ds(c * 16, 16)] = pv[0][c]
                    else:
                        ost_v[buf, m, pl.ds(c * 16, 16)] = pv[m][c] - pv[m - 1][c]

            pltpu.async_copy(ost_v.at[buf], out_hbm.at[b], osem)
            return carry

        lax.fori_loop(0, BPW, batch_body, 0)

        pltpu.make_async_copy(ost_v.at[0], out_hbm.at[base], osem).wait()
        pltpu.make_async_copy(ost_v.at[1], out_hbm.at[base + 1], osem).wait()

    return sc_kernel


def kernel(input, per_sample_weights, offsets, weight):
    B, N = input.shape
    M = offsets.shape[1]
    V, D = weight.shape
    NW = 32
    BPW = B // NW
    NP = N + 8

    idx3 = input.astype(jnp.int32).reshape(B, 2, N // 2)
    psw_p = jnp.pad(per_sample_weights.astype(jnp.float32), ((0, 0), (0, 8)))
    pswx = jnp.broadcast_to(psw_p[:, :, None], (B, NP, 16)).reshape(-1)
    off = offsets.astype(jnp.int32)
    bounds = jnp.concatenate(
        [jnp.zeros((B, 1), jnp.int32), off + 1,
         jnp.zeros((B, 32 - M - 1), jnp.int32)], axis=1).reshape(-1)

    sc = _make_sc_kernel(B, N, M, V, D, NW, BPW)
    return sc(idx3, pswx, bounds, weight)
